# Initial kernel scaffold; baseline (speedup 1.0000x reference)
import jax
import jax.numpy as jnp
from jax.experimental import pallas as pl


def kernel(node_confidences, visual_features, W1, b1, W2, b2, W3, b3, We1, be1, We2, be2):
    raise NotImplementedError("write your pallas kernel here")


# speedup vs baseline: 72.1324x; 72.1324x over previous
import jax
import jax.numpy as jnp
from jax.experimental import pallas as pl
from jax.experimental.pallas import tpu as pltpu

N = 256


def _grid_body(conf_col_ref, conf_row_ref, w1a_ref, w1bt_ref, b1_ref,
               w2_ref, b2_ref, w3_ref, b3_ref,
               we1_ref, be1_ref, we2_ref, be2_ref,
               ce_ref, c0_ref, c1_ref, edges_ref):
    a = w1a_ref[...] + b1_ref[...]
    bt = w1bt_ref[...]

    acc0 = jnp.zeros((N, N), jnp.float32)
    acc1 = jnp.zeros((N, N), jnp.float32)
    acc2 = jnp.zeros((N, N), jnp.float32)
    acc3 = jnp.zeros((N, N), jnp.float32)
    for k in range(32):
        t = jnp.maximum(a[:, k:k + 1] + bt[k:k + 1, :], 0.0)
        acc0 = acc0 + t * w2_ref[k, 0]
        acc1 = acc1 + t * w2_ref[k, 1]
        acc2 = acc2 + t * w2_ref[k, 2]
        acc3 = acc3 + t * w2_ref[k, 3]

    ce_sum = (jnp.maximum(acc0 + b2_ref[0, 0], 0.0) * w3_ref[0, 0]
              + jnp.maximum(acc1 + b2_ref[0, 1], 0.0) * w3_ref[1, 0]
              + jnp.maximum(acc2 + b2_ref[0, 2], 0.0) * w3_ref[2, 0]
              + jnp.maximum(acc3 + b2_ref[0, 3], 0.0) * w3_ref[3, 0])
    ce_grid = jax.nn.sigmoid(ce_sum + b3_ref[0, 0])

    conf_col = conf_col_ref[...]
    conf_row = conf_row_ref[...]

    edge_sum = jnp.zeros((N, N), jnp.float32)
    for m in range(4):
        em = jnp.maximum(conf_col * we1_ref[0, m]
                         + conf_row * we1_ref[1, m]
                         + ce_grid * we1_ref[2, m]
                         + be1_ref[0, m], 0.0)
        edge_sum = edge_sum + em * we2_ref[m, 0]
    edge_grid = jax.nn.sigmoid(edge_sum + be2_ref[0, 0])

    rows = jax.lax.broadcasted_iota(jnp.int32, (N, N), 0)
    cols = jax.lax.broadcasted_iota(jnp.int32, (N, N), 1)
    edges_ref[...] = jnp.where(rows == cols, 0.0, edge_grid)

    keep_left = (jax.lax.broadcasted_iota(jnp.int32, (N, N - 1), 1)
                 < jax.lax.broadcasted_iota(jnp.int32, (N, N - 1), 0))
    ce_ref[...] = jnp.where(keep_left, ce_grid[:, :N - 1], ce_grid[:, 1:])
    c0_ref[...] = jnp.broadcast_to(conf_col, (N, N - 1))
    conf_j_grid = jnp.broadcast_to(conf_row, (N, N))
    c1_ref[...] = jnp.where(keep_left, conf_j_grid[:, :N - 1], conf_j_grid[:, 1:])


def kernel(node_confidences, visual_features, W1, b1, W2, b2, W3, b3,
           We1, be1, We2, be2):
    del visual_features
    conf_col = node_confidences.reshape(N, 1)
    conf_row = node_confidences.reshape(1, N)
    w1a = W1[:N, :]
    w1bt = W1[N:, :].T

    smem = pl.BlockSpec(memory_space=pltpu.SMEM)
    vmem = pl.BlockSpec(memory_space=pltpu.VMEM)
    ce_comp, c0, c1, edges = pl.pallas_call(
        _grid_body,
        out_shape=[
            jax.ShapeDtypeStruct((N, N - 1), jnp.float32),
            jax.ShapeDtypeStruct((N, N - 1), jnp.float32),
            jax.ShapeDtypeStruct((N, N - 1), jnp.float32),
            jax.ShapeDtypeStruct((N, N), jnp.float32),
        ],
        in_specs=[vmem, vmem, vmem, vmem, vmem,
                  smem, smem, smem, smem, smem, smem, smem, smem],
        out_specs=[vmem, vmem, vmem, vmem],
    )(conf_col, conf_row, w1a, w1bt, b1.reshape(1, 32),
      W2, b2.reshape(1, 4), W3, b3.reshape(1, 1),
      We1, be1.reshape(1, 4), We2, be2.reshape(1, 1))

    class_embedding = ce_comp.reshape(-1, 1)
    X = jnp.stack([c0, c1, ce_comp], axis=-1).reshape(-1, 3)
    return (class_embedding, X, edges)
